# baseline (speedup 1.0000x reference)
import functools

import jax
import jax.numpy as jnp
from jax import lax
from jax.experimental import pallas as pl
from jax.experimental.pallas import tpu as pltpu
from jax.experimental.pallas import tpu_sc as plsc

EMB = 64
NC = 2
NS = 16
NW = NC * NS
LPAD = 56
NBUF = 8


def _gather_kernel(bsz, seq):
    per_w = bsz // NW
    mesh = plsc.VectorSubcoreMesh(
        core_axis_name="c", subcore_axis_name="s", num_cores=NC, num_subcores=NS
    )

    @functools.partial(
        pl.kernel,
        out_type=jax.ShapeDtypeStruct((bsz, seq, EMB), jnp.float32),
        mesh=mesh,
        scratch_types=[
            pltpu.VMEM((per_w * LPAD,), jnp.int32),
            pltpu.VMEM((NBUF, seq, EMB), jnp.float32),
            pltpu.SemaphoreType.DMA,
            pltpu.SemaphoreType.DMA,
        ],
        compiler_params=pltpu.CompilerParams(use_tc_tiling_on_sc=False),
    )
    def body(ids_hbm, table_hbm, out_hbm, idx_v, rows_v, gsem, ssem):
        wid = lax.axis_index("s") * NC + lax.axis_index("c")
        base = wid * per_w
        pltpu.sync_copy(ids_hbm.at[wid], idx_v)

        def group(g, _):
            j0 = g * NBUF
            for b in range(NBUF):
                @pl.when(g > 0)
                def _():
                    pltpu.make_async_copy(
                        rows_v.at[b], out_hbm.at[base + j0 + b], ssem
                    ).wait()

                pltpu.async_copy(
                    table_hbm.at[idx_v.at[pl.ds((j0 + b) * LPAD, seq)]],
                    rows_v.at[b],
                    gsem,
                )
            for b in range(NBUF):
                pltpu.make_async_copy(
                    table_hbm.at[idx_v.at[pl.ds((j0 + b) * LPAD, seq)]],
                    rows_v.at[b],
                    gsem,
                ).wait()
                pltpu.async_copy(
                    rows_v.at[b], out_hbm.at[base + j0 + b], ssem
                )
            return 0

        lax.fori_loop(0, per_w // NBUF, group, 0)
        for b in range(NBUF):
            pltpu.make_async_copy(
                rows_v.at[b], out_hbm.at[base + b], ssem
            ).wait()

    return body


def kernel(input_ids, embedding_table):
    bsz, seq = input_ids.shape
    ids = jnp.pad(input_ids.astype(jnp.int32), ((0, 0), (0, LPAD - seq)))
    ids = ids.reshape(NW, (bsz // NW) * LPAD)
    return _gather_kernel(bsz, seq)(ids, embedding_table)
